# baseline (speedup 1.0000x reference)
import functools
import math

import jax
import jax.numpy as jnp
from jax import lax
from jax.experimental import pallas as pl
from jax.experimental.pallas import tpu as pltpu
from jax.experimental.pallas import tpu_sc as plsc

D = 1024
V = 50258
NH = 2
DH = D // NH
NHID = 200
NP = 64
TOPK = 8
S = 2048

NEG_INF = float("-inf")


def _weights_kernel(cw_ref, nu_ref, ro_ref, w_ref):
    cw = cw_ref[...]
    nu = nu_ref[...]
    mu = jnp.mean(cw)
    std = jnp.sqrt(jnp.mean((cw - mu) ** 2)) + 1e-7
    scores = cw + nu * std
    iota = jax.lax.broadcasted_iota(jnp.int32, (1, NP), 1)
    iota8 = jax.lax.broadcasted_iota(jnp.int32, (1, TOPK), 1)
    vals = jnp.zeros((1, TOPK), jnp.float32)
    for i in range(TOPK):
        m = jnp.max(scores)
        idx = jnp.min(jnp.where(scores == m, iota, NP))
        vals = jnp.where(iota8 == i, m, vals)
        scores = jnp.where(iota == idx, NEG_INF, scores)
    ro = ro_ref[...]
    joining = ro == 0
    masked = jnp.where(joining, vals, NEG_INF)
    mx = jnp.max(masked)
    e = jnp.exp(masked - mx)
    w = e / jnp.sum(e)
    w_ref[...] = jnp.where(joining, w, 0.0)


def _joining_weights(chain_weights, noise_unit, return_ops):
    return pl.pallas_call(
        _weights_kernel,
        out_shape=jax.ShapeDtypeStruct((1, TOPK), jnp.float32),
    )(
        chain_weights.reshape(1, NP),
        noise_unit.reshape(1, NP),
        return_ops.reshape(1, TOPK),
    )


def _combine_kernel(w_ref, resp_ref, out_ref):
    acc = w_ref[0, 0] * resp_ref[0]
    for k in range(1, TOPK):
        acc = acc + w_ref[0, k] * resp_ref[k]
    out_ref[...] = acc


def _combine(w, responses, ts=512):
    return pl.pallas_call(
        _combine_kernel,
        grid=(S // ts,),
        in_specs=[
            pl.BlockSpec((1, TOPK), lambda i: (0, 0)),
            pl.BlockSpec((TOPK, ts, D), lambda i: (0, i, 0)),
        ],
        out_specs=pl.BlockSpec((ts, D), lambda i: (i, 0)),
        out_shape=jax.ShapeDtypeStruct((S, D), jnp.float32),
    )(w, responses)


def _matmul_t(a, b, prec=jnp.float32):
    return jax.lax.dot_general(
        a, b, (((1,), (1,)), ((), ())), preferred_element_type=prec
    )


def _qkv_kernel(x_ref, w_ref, b_ref, out_ref):
    out_ref[...] = (_matmul_t(x_ref[...], w_ref[...]) + b_ref[...]).astype(
        jnp.bfloat16
    )


def _qkv(x, w, b, ts=512, tn=1024):
    n = w.shape[0]
    return pl.pallas_call(
        _qkv_kernel,
        grid=(S // ts, n // tn),
        in_specs=[
            pl.BlockSpec((ts, D), lambda i, j: (i, 0)),
            pl.BlockSpec((tn, D), lambda i, j: (j, 0)),
            pl.BlockSpec((1, tn), lambda i, j: (0, j)),
        ],
        out_specs=pl.BlockSpec((ts, tn), lambda i, j: (i, j)),
        out_shape=jax.ShapeDtypeStruct((S, n), jnp.bfloat16),
    )(x, w, b)


def _attn_kernel(q_ref, k_ref, v_ref, out_ref):
    q = q_ref[...]
    k = k_ref[...]
    s = jax.lax.dot_general(
        q, k, (((1,), (1,)), ((), ())), preferred_element_type=jnp.float32
    ) * (1.0 / math.sqrt(DH))
    m = jnp.max(s, axis=1, keepdims=True)
    p = jnp.exp(s - m)
    l = jnp.sum(p, axis=1, keepdims=True)
    out_ref[...] = jnp.dot(
        (p / l).astype(jnp.bfloat16), v_ref[...],
        preferred_element_type=jnp.float32,
    ).astype(jnp.bfloat16)


def _attention(qkv, ts=512):
    return pl.pallas_call(
        _attn_kernel,
        grid=(NH, S // ts),
        in_specs=[
            pl.BlockSpec((ts, DH), lambda h, i: (i, h)),
            pl.BlockSpec((S, DH), lambda h, i: (0, NH + h)),
            pl.BlockSpec((S, DH), lambda h, i: (0, 2 * NH + h)),
        ],
        out_specs=pl.BlockSpec((ts, DH), lambda h, i: (i, h)),
        out_shape=jax.ShapeDtypeStruct((S, D), jnp.bfloat16),
    )(qkv, qkv, qkv)


def _ln(x, w, b):
    mu = jnp.mean(x, axis=-1, keepdims=True)
    var = jnp.mean((x - mu) ** 2, axis=-1, keepdims=True)
    return (x - mu) * jax.lax.rsqrt(var + 1e-5) * w + b


def _post_kernel(
    a_ref, x_ref, wo_ref, bo_ref, ln1w_ref, ln1b_ref,
    w1_ref, b1_ref, w2_ref, b2_ref, ln2w_ref, ln2b_ref, out_ref
):
    o = _matmul_t(a_ref[...], wo_ref[...]) + bo_ref[...]
    x = _ln(x_ref[...] + o, ln1w_ref[...], ln1b_ref[...])
    h = jnp.maximum(_matmul_t(x, w1_ref[...]) + b1_ref[...], 0.0)
    f = _matmul_t(h, w2_ref[...]) + b2_ref[...]
    out_ref[...] = _ln(x + f, ln2w_ref[...], ln2b_ref[...])


def _post(attn_out, x, p, ts=512):
    full = lambda shape: pl.BlockSpec(shape, lambda i: tuple(0 for _ in shape))
    return pl.pallas_call(
        _post_kernel,
        grid=(S // ts,),
        in_specs=[
            pl.BlockSpec((ts, D), lambda i: (i, 0)),
            pl.BlockSpec((ts, D), lambda i: (i, 0)),
            full((D, D)),
            full((1, D)),
            full((1, D)),
            full((1, D)),
            full((NHID, D)),
            full((1, NHID)),
            full((D, NHID)),
            full((1, D)),
            full((1, D)),
            full((1, D)),
        ],
        out_specs=pl.BlockSpec((ts, D), lambda i: (i, 0)),
        out_shape=jax.ShapeDtypeStruct((S, D), jnp.float32),
    )(
        attn_out, x, p["out_proj_w"].astype(jnp.bfloat16),
        p["out_proj_b"].reshape(1, D),
        p["ln1_w"].reshape(1, D), p["ln1_b"].reshape(1, D),
        p["lin1_w"], p["lin1_b"].reshape(1, NHID),
        p["lin2_w"], p["lin2_b"].reshape(1, D),
        p["ln2_w"].reshape(1, D), p["ln2_b"].reshape(1, D),
    )


def _sc_gather_rows(table, idx):
    info = plsc.get_sparse_core_info()
    nc, ns = info.num_cores, info.num_subcores
    nw = nc * ns
    b = idx.shape[0]
    bw = b // nw
    mesh = plsc.VectorSubcoreMesh(core_axis_name="c", subcore_axis_name="s")

    @functools.partial(
        pl.kernel,
        mesh=mesh,
        out_type=jax.ShapeDtypeStruct((b, D), jnp.float32),
        scratch_types=[
            pltpu.VMEM((bw,), jnp.int32),
            pltpu.VMEM((bw, D), jnp.float32),
            pltpu.SemaphoreType.DMA,
        ],
    )
    def gather_kernel(table_hbm, idx_hbm, out_hbm, idx_v, rows_v, sem):
        wid = lax.axis_index("s") * nc + lax.axis_index("c")
        base = wid * bw
        pltpu.sync_copy(idx_hbm.at[pl.ds(base, bw)], idx_v)
        pltpu.async_copy(table_hbm.at[idx_v], rows_v, sem).wait()
        pltpu.sync_copy(rows_v, out_hbm.at[pl.ds(base, bw)])

    return gather_kernel(table, idx)


_SB = S // 128


def _decoder_kernel(
    x3_ref, w_ref, rows3_ref, out_ref, loss_ref, m_ref, s_ref, ll_ref,
    *, tv, nvt
):
    j = pl.program_id(0)

    @pl.when(j == 0)
    def _init():
        m_ref[...] = jnp.full_like(m_ref, NEG_INF)
        s_ref[...] = jnp.zeros_like(s_ref)
        x32 = x3_ref[...].astype(jnp.float32)
        w32 = rows3_ref[...].astype(jnp.bfloat16).astype(jnp.float32)
        ll_ref[...] = jnp.sum(x32 * w32, axis=2)

    w16 = w_ref[...].astype(jnp.bfloat16)
    for sb in range(_SB):
        out_ref[:, sb, :] = jax.lax.dot_general(
            w16, x3_ref[sb], (((1,), (1,)), ((), ())),
            preferred_element_type=jnp.float32,
        )
    lt = out_ref[...]

    def update(lm):
        tile_max = jnp.max(lm, axis=0)
        m_old = m_ref[...]
        m_new = jnp.maximum(m_old, tile_max)
        s_ref[...] = s_ref[...] * jnp.exp(m_old - m_new) + jnp.sum(
            jnp.exp(lm - m_new[None]), axis=0
        )
        m_ref[...] = m_new

    @pl.when(j < nvt - 1)
    def _mid():
        update(lt)

    @pl.when(j == nvt - 1)
    def _fin():
        viota = jax.lax.broadcasted_iota(jnp.int32, (tv, _SB, 128), 0)
        update(jnp.where(j * tv + viota < V, lt, NEG_INF))
        sidx = jax.lax.broadcasted_iota(jnp.int32, (_SB, 128), 0) * 128 + \
            jax.lax.broadcasted_iota(jnp.int32, (_SB, 128), 1)
        nll = (m_ref[...] + jnp.log(s_ref[...])) - ll_ref[...]
        nll = jnp.where(sidx < S - 1, nll, 0.0)
        loss_ref[0, 0] = jnp.sum(nll) * (1.0 / (S - 1))


def _decoder_loss(x3, w, rows3, tv=512):
    nvt = (V + tv - 1) // tv
    out, loss = pl.pallas_call(
        functools.partial(_decoder_kernel, tv=tv, nvt=nvt),
        grid=(nvt,),
        in_specs=[
            pl.BlockSpec((_SB, 128, D), lambda j: (0, 0, 0)),
            pl.BlockSpec((tv, D), lambda j: (j, 0)),
            pl.BlockSpec((_SB, 128, D), lambda j: (0, 0, 0)),
        ],
        compiler_params=pltpu.CompilerParams(
            dimension_semantics=("arbitrary",),
        ),
        out_specs=[
            pl.BlockSpec((tv, _SB, 128), lambda j: (j, 0, 0)),
            pl.BlockSpec(memory_space=pltpu.SMEM),
        ],
        out_shape=[
            jax.ShapeDtypeStruct((V, _SB, 128), jnp.float32),
            jax.ShapeDtypeStruct((1, 1), jnp.float32),
        ],
        scratch_shapes=[
            pltpu.VMEM((_SB, 128), jnp.float32),
            pltpu.VMEM((_SB, 128), jnp.float32),
            pltpu.VMEM((_SB, 128), jnp.float32),
        ],
    )(x3, w, rows3)
    return out, loss[0, 0]


def kernel(inputs, chain_weights, noise_unit, responses, return_ops, params,
           decoder_w):
    labels = jnp.concatenate([inputs[0, 1:], jnp.zeros((1,), jnp.int32)])
    lab_rows = _sc_gather_rows(decoder_w, labels)

    w = _joining_weights(chain_weights, noise_unit, return_ops)
    x = _combine(w, responses.reshape(TOPK, S, D))

    for p in params["layers"]:
        qkv = _qkv(
            x.astype(jnp.bfloat16),
            p["in_proj_w"].astype(jnp.bfloat16),
            p["in_proj_b"].reshape(1, 3 * D),
        )
        attn_out = _attention(qkv)
        x = _post(attn_out, x, p)

    x3 = x.astype(jnp.bfloat16).reshape(_SB, 128, D)
    rows3 = lab_rows.reshape(_SB, 128, D)
    out3, loss = _decoder_loss(x3, decoder_w, rows3)
    outputs = out3.reshape(V, S).T.reshape(1, S, V)
    return (outputs, loss)


# speedup vs baseline: 2.2307x; 2.2307x over previous
import functools
import math

import jax
import jax.numpy as jnp
from jax import lax
from jax.experimental import pallas as pl
from jax.experimental.pallas import tpu as pltpu
from jax.experimental.pallas import tpu_sc as plsc

D = 1024
V = 50258
NH = 2
DH = D // NH
NHID = 200
NP = 64
TOPK = 8
S = 2048

NEG_INF = float("-inf")


def _weights_kernel(cw_ref, nu_ref, ro_ref, w_ref):
    cw = cw_ref[...]
    nu = nu_ref[...]
    mu = jnp.mean(cw)
    std = jnp.sqrt(jnp.mean((cw - mu) ** 2)) + 1e-7
    scores = cw + nu * std
    iota = jax.lax.broadcasted_iota(jnp.int32, (1, NP), 1)
    iota8 = jax.lax.broadcasted_iota(jnp.int32, (1, TOPK), 1)
    vals = jnp.zeros((1, TOPK), jnp.float32)
    for i in range(TOPK):
        m = jnp.max(scores)
        idx = jnp.min(jnp.where(scores == m, iota, NP))
        vals = jnp.where(iota8 == i, m, vals)
        scores = jnp.where(iota == idx, NEG_INF, scores)
    ro = ro_ref[...]
    joining = ro == 0
    masked = jnp.where(joining, vals, NEG_INF)
    mx = jnp.max(masked)
    e = jnp.exp(masked - mx)
    w = e / jnp.sum(e)
    w_ref[...] = jnp.where(joining, w, 0.0)


def _joining_weights(chain_weights, noise_unit, return_ops):
    return pl.pallas_call(
        _weights_kernel,
        out_shape=jax.ShapeDtypeStruct((1, TOPK), jnp.float32),
    )(
        chain_weights.reshape(1, NP),
        noise_unit.reshape(1, NP),
        return_ops.reshape(1, TOPK),
    )


def _combine_kernel(w_ref, resp_ref, out_ref):
    acc = w_ref[0, 0] * resp_ref[0]
    for k in range(1, TOPK):
        acc = acc + w_ref[0, k] * resp_ref[k]
    out_ref[...] = acc


def _combine(w, responses, ts=512):
    return pl.pallas_call(
        _combine_kernel,
        grid=(S // ts,),
        in_specs=[
            pl.BlockSpec((1, TOPK), lambda i: (0, 0)),
            pl.BlockSpec((TOPK, ts, D), lambda i: (0, i, 0)),
        ],
        out_specs=pl.BlockSpec((ts, D), lambda i: (i, 0)),
        out_shape=jax.ShapeDtypeStruct((S, D), jnp.float32),
    )(w, responses)


def _matmul_t(a, b, prec=jnp.float32):
    return jax.lax.dot_general(
        a, b, (((1,), (1,)), ((), ())), preferred_element_type=prec
    )


def _qkv_kernel(x_ref, w_ref, b_ref, out_ref):
    out_ref[...] = _matmul_t(x_ref[...], w_ref[...]) + b_ref[...]


def _qkv(x, w, b, ts=512, tn=1024):
    n = w.shape[0]
    return pl.pallas_call(
        _qkv_kernel,
        grid=(S // ts, n // tn),
        in_specs=[
            pl.BlockSpec((ts, D), lambda i, j: (i, 0)),
            pl.BlockSpec((tn, D), lambda i, j: (j, 0)),
            pl.BlockSpec((1, tn), lambda i, j: (0, j)),
        ],
        out_specs=pl.BlockSpec((ts, tn), lambda i, j: (i, j)),
        out_shape=jax.ShapeDtypeStruct((S, n), jnp.float32),
    )(x, w, b)


def _attn_kernel(q_ref, k_ref, v_ref, out_ref):
    q = q_ref[...]
    k = k_ref[...]
    s = jax.lax.dot_general(
        q, k, (((1,), (1,)), ((), ())), preferred_element_type=jnp.float32
    ) * (1.0 / math.sqrt(DH))
    m = jnp.max(s, axis=1, keepdims=True)
    p = jnp.exp(s - m)
    l = jnp.sum(p, axis=1, keepdims=True)
    out_ref[...] = jnp.dot(
        p / l, v_ref[...], preferred_element_type=jnp.float32
    )


def _attention(qkv, ts=512):
    return pl.pallas_call(
        _attn_kernel,
        grid=(NH, S // ts),
        in_specs=[
            pl.BlockSpec((ts, DH), lambda h, i: (i, h)),
            pl.BlockSpec((S, DH), lambda h, i: (0, NH + h)),
            pl.BlockSpec((S, DH), lambda h, i: (0, 2 * NH + h)),
        ],
        out_specs=pl.BlockSpec((ts, DH), lambda h, i: (i, h)),
        out_shape=jax.ShapeDtypeStruct((S, D), jnp.float32),
    )(qkv, qkv, qkv)


def _ln(x, w, b):
    mu = jnp.mean(x, axis=-1, keepdims=True)
    var = jnp.mean((x - mu) ** 2, axis=-1, keepdims=True)
    return (x - mu) * jax.lax.rsqrt(var + 1e-5) * w + b


def _post_kernel(
    a_ref, x_ref, wo_ref, bo_ref, ln1w_ref, ln1b_ref,
    w1_ref, b1_ref, w2_ref, b2_ref, ln2w_ref, ln2b_ref, out_ref
):
    o = _matmul_t(a_ref[...], wo_ref[...]) + bo_ref[...]
    x = _ln(x_ref[...] + o, ln1w_ref[...], ln1b_ref[...])
    h = jnp.maximum(_matmul_t(x, w1_ref[...]) + b1_ref[...], 0.0)
    f = _matmul_t(h, w2_ref[...]) + b2_ref[...]
    out_ref[...] = _ln(x + f, ln2w_ref[...], ln2b_ref[...])


def _post(attn_out, x, p, ts=512):
    full = lambda shape: pl.BlockSpec(shape, lambda i: tuple(0 for _ in shape))
    return pl.pallas_call(
        _post_kernel,
        grid=(S // ts,),
        in_specs=[
            pl.BlockSpec((ts, D), lambda i: (i, 0)),
            pl.BlockSpec((ts, D), lambda i: (i, 0)),
            full((D, D)),
            full((1, D)),
            full((1, D)),
            full((1, D)),
            full((NHID, D)),
            full((1, NHID)),
            full((D, NHID)),
            full((1, D)),
            full((1, D)),
            full((1, D)),
        ],
        out_specs=pl.BlockSpec((ts, D), lambda i: (i, 0)),
        out_shape=jax.ShapeDtypeStruct((S, D), jnp.float32),
    )(
        attn_out, x, p["out_proj_w"],
        p["out_proj_b"].reshape(1, D),
        p["ln1_w"].reshape(1, D), p["ln1_b"].reshape(1, D),
        p["lin1_w"], p["lin1_b"].reshape(1, NHID),
        p["lin2_w"], p["lin2_b"].reshape(1, D),
        p["ln2_w"].reshape(1, D), p["ln2_b"].reshape(1, D),
    )


def _sc_gather_rows(table, idx):
    info = plsc.get_sparse_core_info()
    nc, ns = info.num_cores, info.num_subcores
    nw = nc * ns
    b = idx.shape[0]
    bw = b // nw
    mesh = plsc.VectorSubcoreMesh(core_axis_name="c", subcore_axis_name="s")

    @functools.partial(
        pl.kernel,
        mesh=mesh,
        out_type=jax.ShapeDtypeStruct((b, D), jnp.float32),
        scratch_types=[
            pltpu.VMEM((bw,), jnp.int32),
            pltpu.VMEM((bw, D), jnp.float32),
            pltpu.SemaphoreType.DMA,
        ],
    )
    def gather_kernel(table_hbm, idx_hbm, out_hbm, idx_v, rows_v, sem):
        wid = lax.axis_index("s") * nc + lax.axis_index("c")
        base = wid * bw
        pltpu.sync_copy(idx_hbm.at[pl.ds(base, bw)], idx_v)
        pltpu.async_copy(table_hbm.at[idx_v], rows_v, sem).wait()
        pltpu.sync_copy(rows_v, out_hbm.at[pl.ds(base, bw)])

    return gather_kernel(table, idx)


def _decoder_kernel(
    x_ref, w_ref, rows_ref, out_ref, loss_ref, m_ref, s_ref, ll_ref, *, tv, nvt
):
    j = pl.program_id(0)

    @pl.when(j == 0)
    def _init():
        m_ref[...] = jnp.full_like(m_ref, NEG_INF)
        s_ref[...] = jnp.zeros_like(s_ref)
        x32 = x_ref[...].astype(jnp.float32)
        w32 = rows_ref[...].astype(jnp.bfloat16).astype(jnp.float32)
        ll_ref[...] = jnp.sum(x32 * w32, axis=1, keepdims=True)

    logits = _matmul_t(x_ref[...], w_ref[...].astype(jnp.bfloat16))
    out_ref[...] = logits

    def update(lm):
        tile_max = jnp.max(lm, axis=1, keepdims=True)
        m_old = m_ref[...]
        m_new = jnp.maximum(m_old, tile_max)
        s_ref[...] = s_ref[...] * jnp.exp(m_old - m_new) + jnp.sum(
            jnp.exp(lm - m_new), axis=1, keepdims=True
        )
        m_ref[...] = m_new

    @pl.when(j < nvt - 1)
    def _mid():
        update(logits)

    @pl.when(j == nvt - 1)
    def _fin():
        iota = jax.lax.broadcasted_iota(jnp.int32, (S, tv), 1)
        update(jnp.where(j * tv + iota < V, logits, NEG_INF))
        row = jax.lax.broadcasted_iota(jnp.int32, (S, 1), 0)
        nll = (m_ref[...] + jnp.log(s_ref[...])) - ll_ref[...]
        nll = jnp.where(row < S - 1, nll, 0.0)
        loss_ref[0, 0] = jnp.sum(nll) * (1.0 / (S - 1))


def _decoder_loss(x, w, rows, tv=1024):
    nvt = (V + tv - 1) // tv
    out, loss = pl.pallas_call(
        functools.partial(_decoder_kernel, tv=tv, nvt=nvt),
        grid=(nvt,),
        in_specs=[
            pl.BlockSpec((S, D), lambda j: (0, 0)),
            pl.BlockSpec((tv, D), lambda j: (j, 0)),
            pl.BlockSpec((S, D), lambda j: (0, 0)),
        ],
        compiler_params=pltpu.CompilerParams(
            dimension_semantics=("arbitrary",),
        ),
        out_specs=[
            pl.BlockSpec((S, tv), lambda j: (0, j)),
            pl.BlockSpec(memory_space=pltpu.SMEM),
        ],
        out_shape=[
            jax.ShapeDtypeStruct((S, V), jnp.float32),
            jax.ShapeDtypeStruct((1, 1), jnp.float32),
        ],
        scratch_shapes=[
            pltpu.VMEM((S, 1), jnp.float32),
            pltpu.VMEM((S, 1), jnp.float32),
            pltpu.VMEM((S, 1), jnp.float32),
        ],
    )(x, w, rows)
    return out, loss[0, 0]


def kernel(inputs, chain_weights, noise_unit, responses, return_ops, params,
           decoder_w):
    labels = jnp.concatenate([inputs[0, 1:], jnp.zeros((1,), jnp.int32)])
    lab_rows = _sc_gather_rows(decoder_w, labels)

    w = _joining_weights(chain_weights, noise_unit, return_ops)
    x = _combine(w, responses.reshape(TOPK, S, D))

    for p in params["layers"]:
        qkv = _qkv(
            x,
            p["in_proj_w"],
            p["in_proj_b"].reshape(1, 3 * D),
        )
        attn_out = _attention(qkv)
        x = _post(attn_out, x, p)

    outputs, loss = _decoder_loss(x.astype(jnp.bfloat16), decoder_w, lab_rows)
    return (outputs.reshape(1, S, V), loss)


# speedup vs baseline: 2.2740x; 1.0194x over previous
import functools
import math

import jax
import jax.numpy as jnp
from jax import lax
from jax.experimental import pallas as pl
from jax.experimental.pallas import tpu as pltpu
from jax.experimental.pallas import tpu_sc as plsc

D = 1024
V = 50258
NH = 2
DH = D // NH
NHID = 200
NP = 64
TOPK = 8
S = 2048

NEG_INF = float("-inf")


def _weights_kernel(cw_ref, nu_ref, ro_ref, w_ref):
    cw = cw_ref[...]
    nu = nu_ref[...]
    mu = jnp.mean(cw)
    std = jnp.sqrt(jnp.mean((cw - mu) ** 2)) + 1e-7
    scores = cw + nu * std
    iota = jax.lax.broadcasted_iota(jnp.int32, (1, NP), 1)
    iota8 = jax.lax.broadcasted_iota(jnp.int32, (1, TOPK), 1)
    vals = jnp.zeros((1, TOPK), jnp.float32)
    for i in range(TOPK):
        m = jnp.max(scores)
        idx = jnp.min(jnp.where(scores == m, iota, NP))
        vals = jnp.where(iota8 == i, m, vals)
        scores = jnp.where(iota == idx, NEG_INF, scores)
    ro = ro_ref[...]
    joining = ro == 0
    masked = jnp.where(joining, vals, NEG_INF)
    mx = jnp.max(masked)
    e = jnp.exp(masked - mx)
    w = e / jnp.sum(e)
    w_ref[...] = jnp.where(joining, w, 0.0)


def _joining_weights(chain_weights, noise_unit, return_ops):
    return pl.pallas_call(
        _weights_kernel,
        out_shape=jax.ShapeDtypeStruct((1, TOPK), jnp.float32),
    )(
        chain_weights.reshape(1, NP),
        noise_unit.reshape(1, NP),
        return_ops.reshape(1, TOPK),
    )


def _combine_kernel(w_ref, resp_ref, out_ref):
    acc = w_ref[0, 0] * resp_ref[0]
    for k in range(1, TOPK):
        acc = acc + w_ref[0, k] * resp_ref[k]
    out_ref[...] = acc


def _combine(w, responses, ts=512):
    return pl.pallas_call(
        _combine_kernel,
        grid=(S // ts,),
        in_specs=[
            pl.BlockSpec((1, TOPK), lambda i: (0, 0)),
            pl.BlockSpec((TOPK, ts, D), lambda i: (0, i, 0)),
        ],
        out_specs=pl.BlockSpec((ts, D), lambda i: (i, 0)),
        out_shape=jax.ShapeDtypeStruct((S, D), jnp.float32),
    )(w, responses)


def _matmul_t(a, b, prec=jnp.float32):
    return jax.lax.dot_general(
        a, b, (((1,), (1,)), ((), ())), preferred_element_type=prec
    )


def _qkv_kernel(x_ref, w_ref, b_ref, out_ref):
    out_ref[...] = _matmul_t(x_ref[...], w_ref[...]) + b_ref[...]


def _qkv(x, w, b, ts=1024, tn=1024):
    n = w.shape[0]
    return pl.pallas_call(
        _qkv_kernel,
        grid=(S // ts, n // tn),
        in_specs=[
            pl.BlockSpec((ts, D), lambda i, j: (i, 0)),
            pl.BlockSpec((tn, D), lambda i, j: (j, 0)),
            pl.BlockSpec((1, tn), lambda i, j: (0, j)),
        ],
        out_specs=pl.BlockSpec((ts, tn), lambda i, j: (i, j)),
        out_shape=jax.ShapeDtypeStruct((S, n), jnp.float32),
    )(x, w, b)


def _attn_kernel(q_ref, k_ref, v_ref, out_ref):
    q = q_ref[...]
    k = k_ref[...]
    s = jax.lax.dot_general(
        q, k, (((1,), (1,)), ((), ())), preferred_element_type=jnp.float32
    ) * (1.0 / math.sqrt(DH))
    m = jnp.max(s, axis=1, keepdims=True)
    p = jnp.exp(s - m)
    l = jnp.sum(p, axis=1, keepdims=True)
    out_ref[...] = jnp.dot(
        p / l, v_ref[...], preferred_element_type=jnp.float32
    )


def _attention(qkv, ts=1024):
    return pl.pallas_call(
        _attn_kernel,
        grid=(NH, S // ts),
        in_specs=[
            pl.BlockSpec((ts, DH), lambda h, i: (i, h)),
            pl.BlockSpec((S, DH), lambda h, i: (0, NH + h)),
            pl.BlockSpec((S, DH), lambda h, i: (0, 2 * NH + h)),
        ],
        out_specs=pl.BlockSpec((ts, DH), lambda h, i: (i, h)),
        out_shape=jax.ShapeDtypeStruct((S, D), jnp.float32),
    )(qkv, qkv, qkv)


def _ln(x, w, b):
    mu = jnp.mean(x, axis=-1, keepdims=True)
    var = jnp.mean((x - mu) ** 2, axis=-1, keepdims=True)
    return (x - mu) * jax.lax.rsqrt(var + 1e-5) * w + b


def _post_kernel(
    a_ref, x_ref, wo_ref, bo_ref, ln1w_ref, ln1b_ref,
    w1_ref, b1_ref, w2_ref, b2_ref, ln2w_ref, ln2b_ref, out_ref
):
    o = _matmul_t(a_ref[...], wo_ref[...]) + bo_ref[...]
    x = _ln(x_ref[...] + o, ln1w_ref[...], ln1b_ref[...])
    h = jnp.maximum(_matmul_t(x, w1_ref[...]) + b1_ref[...], 0.0)
    f = _matmul_t(h, w2_ref[...]) + b2_ref[...]
    out_ref[...] = _ln(x + f, ln2w_ref[...], ln2b_ref[...])


def _post(attn_out, x, p, ts=1024):
    full = lambda shape: pl.BlockSpec(shape, lambda i: tuple(0 for _ in shape))
    return pl.pallas_call(
        _post_kernel,
        grid=(S // ts,),
        in_specs=[
            pl.BlockSpec((ts, D), lambda i: (i, 0)),
            pl.BlockSpec((ts, D), lambda i: (i, 0)),
            full((D, D)),
            full((1, D)),
            full((1, D)),
            full((1, D)),
            full((NHID, D)),
            full((1, NHID)),
            full((D, NHID)),
            full((1, D)),
            full((1, D)),
            full((1, D)),
        ],
        out_specs=pl.BlockSpec((ts, D), lambda i: (i, 0)),
        out_shape=jax.ShapeDtypeStruct((S, D), jnp.float32),
    )(
        attn_out, x, p["out_proj_w"],
        p["out_proj_b"].reshape(1, D),
        p["ln1_w"].reshape(1, D), p["ln1_b"].reshape(1, D),
        p["lin1_w"], p["lin1_b"].reshape(1, NHID),
        p["lin2_w"], p["lin2_b"].reshape(1, D),
        p["ln2_w"].reshape(1, D), p["ln2_b"].reshape(1, D),
    )


def _sc_gather_rows(table, idx):
    info = plsc.get_sparse_core_info()
    nc, ns = info.num_cores, info.num_subcores
    nw = nc * ns
    b = idx.shape[0]
    bw = b // nw
    mesh = plsc.VectorSubcoreMesh(core_axis_name="c", subcore_axis_name="s")

    @functools.partial(
        pl.kernel,
        mesh=mesh,
        out_type=jax.ShapeDtypeStruct((b, D), jnp.float32),
        scratch_types=[
            pltpu.VMEM((bw,), jnp.int32),
            pltpu.VMEM((bw, D), jnp.float32),
            pltpu.SemaphoreType.DMA,
        ],
    )
    def gather_kernel(table_hbm, idx_hbm, out_hbm, idx_v, rows_v, sem):
        wid = lax.axis_index("s") * nc + lax.axis_index("c")
        base = wid * bw
        pltpu.sync_copy(idx_hbm.at[pl.ds(base, bw)], idx_v)
        pltpu.async_copy(table_hbm.at[idx_v], rows_v, sem).wait()
        pltpu.sync_copy(rows_v, out_hbm.at[pl.ds(base, bw)])

    return gather_kernel(table, idx)


def _decoder_kernel(
    x_ref, w_ref, rows_ref, out_ref, loss_ref, m_ref, s_ref, ll_ref, *, tv, nvt
):
    j = pl.program_id(0)

    @pl.when(j == 0)
    def _init():
        m_ref[...] = jnp.full_like(m_ref, NEG_INF)
        s_ref[...] = jnp.zeros_like(s_ref)
        x32 = x_ref[...].astype(jnp.float32)
        w32 = rows_ref[...].astype(jnp.bfloat16).astype(jnp.float32)
        ll_ref[...] = jnp.sum(x32 * w32, axis=1, keepdims=True)

    logits = _matmul_t(x_ref[...], w_ref[...].astype(jnp.bfloat16))
    out_ref[...] = logits

    def update(lm):
        tile_max = jnp.max(lm, axis=1, keepdims=True)
        m_old = m_ref[...]
        m_new = jnp.maximum(m_old, tile_max)
        s_ref[...] = s_ref[...] * jnp.exp(m_old - m_new) + jnp.sum(
            jnp.exp(lm - m_new), axis=1, keepdims=True
        )
        m_ref[...] = m_new

    @pl.when(j < nvt - 1)
    def _mid():
        update(logits)

    @pl.when(j == nvt - 1)
    def _fin():
        iota = jax.lax.broadcasted_iota(jnp.int32, (S, tv), 1)
        update(jnp.where(j * tv + iota < V, logits, NEG_INF))
        row = jax.lax.broadcasted_iota(jnp.int32, (S, 1), 0)
        nll = (m_ref[...] + jnp.log(s_ref[...])) - ll_ref[...]
        nll = jnp.where(row < S - 1, nll, 0.0)
        loss_ref[0, 0] = jnp.sum(nll) * (1.0 / (S - 1))


def _decoder_loss(x, w, rows, tv=1024):
    nvt = (V + tv - 1) // tv
    out, loss = pl.pallas_call(
        functools.partial(_decoder_kernel, tv=tv, nvt=nvt),
        grid=(nvt,),
        in_specs=[
            pl.BlockSpec((S, D), lambda j: (0, 0)),
            pl.BlockSpec((tv, D), lambda j: (j, 0)),
            pl.BlockSpec((S, D), lambda j: (0, 0)),
        ],
        compiler_params=pltpu.CompilerParams(
            dimension_semantics=("arbitrary",),
        ),
        out_specs=[
            pl.BlockSpec((S, tv), lambda j: (0, j)),
            pl.BlockSpec(memory_space=pltpu.SMEM),
        ],
        out_shape=[
            jax.ShapeDtypeStruct((S, V), jnp.float32),
            jax.ShapeDtypeStruct((1, 1), jnp.float32),
        ],
        scratch_shapes=[
            pltpu.VMEM((S, 1), jnp.float32),
            pltpu.VMEM((S, 1), jnp.float32),
            pltpu.VMEM((S, 1), jnp.float32),
        ],
    )(x, w, rows)
    return out, loss[0, 0]


def kernel(inputs, chain_weights, noise_unit, responses, return_ops, params,
           decoder_w):
    labels = jnp.concatenate([inputs[0, 1:], jnp.zeros((1,), jnp.int32)])
    lab_rows = _sc_gather_rows(decoder_w, labels)

    w = _joining_weights(chain_weights, noise_unit, return_ops)
    x = _combine(w, responses.reshape(TOPK, S, D))

    for p in params["layers"]:
        qkv = _qkv(
            x,
            p["in_proj_w"],
            p["in_proj_b"].reshape(1, 3 * D),
        )
        attn_out = _attention(qkv)
        x = _post(attn_out, x, p)

    outputs, loss = _decoder_loss(x.astype(jnp.bfloat16), decoder_w, lab_rows)
    return (outputs.reshape(1, S, V), loss)


# speedup vs baseline: 2.2974x; 1.0103x over previous
import functools
import math

import jax
import jax.numpy as jnp
from jax import lax
from jax.experimental import pallas as pl
from jax.experimental.pallas import tpu as pltpu
from jax.experimental.pallas import tpu_sc as plsc

D = 1024
V = 50258
NH = 2
DH = D // NH
NHID = 200
NP = 64
TOPK = 8
S = 2048

NEG_INF = float("-inf")


def _weights_kernel(cw_ref, nu_ref, ro_ref, w_ref):
    cw = cw_ref[...]
    nu = nu_ref[...]
    mu = jnp.mean(cw)
    std = jnp.sqrt(jnp.mean((cw - mu) ** 2)) + 1e-7
    scores = cw + nu * std
    iota = jax.lax.broadcasted_iota(jnp.int32, (1, NP), 1)
    iota8 = jax.lax.broadcasted_iota(jnp.int32, (1, TOPK), 1)
    vals = jnp.zeros((1, TOPK), jnp.float32)
    for i in range(TOPK):
        m = jnp.max(scores)
        idx = jnp.min(jnp.where(scores == m, iota, NP))
        vals = jnp.where(iota8 == i, m, vals)
        scores = jnp.where(iota == idx, NEG_INF, scores)
    ro = ro_ref[...]
    joining = ro == 0
    masked = jnp.where(joining, vals, NEG_INF)
    mx = jnp.max(masked)
    e = jnp.exp(masked - mx)
    w = e / jnp.sum(e)
    w_ref[...] = jnp.where(joining, w, 0.0)


def _joining_weights(chain_weights, noise_unit, return_ops):
    return pl.pallas_call(
        _weights_kernel,
        out_shape=jax.ShapeDtypeStruct((1, TOPK), jnp.float32),
    )(
        chain_weights.reshape(1, NP),
        noise_unit.reshape(1, NP),
        return_ops.reshape(1, TOPK),
    )


def _combine_kernel(w_ref, resp_ref, out_ref):
    acc = w_ref[0, 0] * resp_ref[0]
    for k in range(1, TOPK):
        acc = acc + w_ref[0, k] * resp_ref[k]
    out_ref[...] = acc


def _combine(w, responses, ts=512):
    return pl.pallas_call(
        _combine_kernel,
        grid=(S // ts,),
        in_specs=[
            pl.BlockSpec((1, TOPK), lambda i: (0, 0)),
            pl.BlockSpec((TOPK, ts, D), lambda i: (0, i, 0)),
        ],
        out_specs=pl.BlockSpec((ts, D), lambda i: (i, 0)),
        out_shape=jax.ShapeDtypeStruct((S, D), jnp.float32),
    )(w, responses)


def _matmul_t(a, b, prec=jnp.float32):
    return jax.lax.dot_general(
        a, b, (((1,), (1,)), ((), ())), preferred_element_type=prec
    )


def _qkv_kernel(x_ref, w_ref, b_ref, out_ref):
    out_ref[...] = _matmul_t(x_ref[...], w_ref[...]) + b_ref[...]


def _qkv(x, w, b, ts=1024, tn=1024):
    n = w.shape[0]
    return pl.pallas_call(
        _qkv_kernel,
        grid=(S // ts, n // tn),
        in_specs=[
            pl.BlockSpec((ts, D), lambda i, j: (i, 0)),
            pl.BlockSpec((tn, D), lambda i, j: (j, 0)),
            pl.BlockSpec((1, tn), lambda i, j: (0, j)),
        ],
        out_specs=pl.BlockSpec((ts, tn), lambda i, j: (i, j)),
        out_shape=jax.ShapeDtypeStruct((S, n), jnp.float32),
    )(x, w, b)


def _attn_kernel(q_ref, k_ref, v_ref, out_ref):
    q = q_ref[...]
    k = k_ref[...]
    s = jax.lax.dot_general(
        q, k, (((1,), (1,)), ((), ())), preferred_element_type=jnp.float32
    ) * (1.0 / math.sqrt(DH))
    m = jnp.max(s, axis=1, keepdims=True)
    p = jnp.exp(s - m)
    l = jnp.sum(p, axis=1, keepdims=True)
    out_ref[...] = jnp.dot(
        p / l, v_ref[...], preferred_element_type=jnp.float32
    )


def _attention(qkv, ts=1024):
    return pl.pallas_call(
        _attn_kernel,
        grid=(NH, S // ts),
        in_specs=[
            pl.BlockSpec((ts, DH), lambda h, i: (i, h)),
            pl.BlockSpec((S, DH), lambda h, i: (0, NH + h)),
            pl.BlockSpec((S, DH), lambda h, i: (0, 2 * NH + h)),
        ],
        out_specs=pl.BlockSpec((ts, DH), lambda h, i: (i, h)),
        out_shape=jax.ShapeDtypeStruct((S, D), jnp.float32),
    )(qkv, qkv, qkv)


def _ln(x, w, b):
    mu = jnp.mean(x, axis=-1, keepdims=True)
    var = jnp.mean((x - mu) ** 2, axis=-1, keepdims=True)
    return (x - mu) * jax.lax.rsqrt(var + 1e-5) * w + b


def _post_kernel(
    a_ref, x_ref, wo_ref, bo_ref, ln1w_ref, ln1b_ref,
    w1_ref, b1_ref, w2_ref, b2_ref, ln2w_ref, ln2b_ref, out_ref
):
    o = _matmul_t(a_ref[...], wo_ref[...]) + bo_ref[...]
    x = _ln(x_ref[...] + o, ln1w_ref[...], ln1b_ref[...])
    h = jnp.maximum(_matmul_t(x, w1_ref[...]) + b1_ref[...], 0.0)
    f = _matmul_t(h, w2_ref[...]) + b2_ref[...]
    out_ref[...] = _ln(x + f, ln2w_ref[...], ln2b_ref[...]).astype(
        out_ref.dtype
    )


def _post(attn_out, x, p, ts=1024, out_dtype=jnp.float32):
    full = lambda shape: pl.BlockSpec(shape, lambda i: tuple(0 for _ in shape))
    return pl.pallas_call(
        _post_kernel,
        grid=(S // ts,),
        in_specs=[
            pl.BlockSpec((ts, D), lambda i: (i, 0)),
            pl.BlockSpec((ts, D), lambda i: (i, 0)),
            full((D, D)),
            full((1, D)),
            full((1, D)),
            full((1, D)),
            full((NHID, D)),
            full((1, NHID)),
            full((D, NHID)),
            full((1, D)),
            full((1, D)),
            full((1, D)),
        ],
        out_specs=pl.BlockSpec((ts, D), lambda i: (i, 0)),
        out_shape=jax.ShapeDtypeStruct((S, D), out_dtype),
    )(
        attn_out, x, p["out_proj_w"],
        p["out_proj_b"].reshape(1, D),
        p["ln1_w"].reshape(1, D), p["ln1_b"].reshape(1, D),
        p["lin1_w"], p["lin1_b"].reshape(1, NHID),
        p["lin2_w"], p["lin2_b"].reshape(1, D),
        p["ln2_w"].reshape(1, D), p["ln2_b"].reshape(1, D),
    )


def _sc_gather_rows(table, idx):
    info = plsc.get_sparse_core_info()
    nc, ns = info.num_cores, info.num_subcores
    nw = nc * ns
    b = idx.shape[0]
    bw = b // nw
    mesh = plsc.VectorSubcoreMesh(core_axis_name="c", subcore_axis_name="s")

    @functools.partial(
        pl.kernel,
        mesh=mesh,
        out_type=jax.ShapeDtypeStruct((b, D), jnp.float32),
        scratch_types=[
            pltpu.VMEM((bw,), jnp.int32),
            pltpu.VMEM((bw, D), jnp.float32),
            pltpu.SemaphoreType.DMA,
        ],
    )
    def gather_kernel(table_hbm, idx_hbm, out_hbm, idx_v, rows_v, sem):
        wid = lax.axis_index("s") * nc + lax.axis_index("c")
        base = wid * bw
        pltpu.sync_copy(idx_hbm.at[pl.ds(base, bw)], idx_v)
        pltpu.async_copy(table_hbm.at[idx_v], rows_v, sem).wait()
        pltpu.sync_copy(rows_v, out_hbm.at[pl.ds(base, bw)])

    return gather_kernel(table, idx)


def _decoder_kernel(
    x_ref, w_ref, rows_ref, out_ref, loss_ref, m_ref, s_ref, ll_ref, *, tv, nvt
):
    j = pl.program_id(0)

    @pl.when(j == 0)
    def _init():
        m_ref[...] = jnp.full_like(m_ref, NEG_INF)
        s_ref[...] = jnp.zeros_like(s_ref)
        x32 = x_ref[...].astype(jnp.float32)
        w32 = rows_ref[...].astype(jnp.bfloat16).astype(jnp.float32)
        ll_ref[...] = jnp.sum(x32 * w32, axis=1, keepdims=True)

    logits = _matmul_t(x_ref[...], w_ref[...].astype(jnp.bfloat16))
    out_ref[...] = logits

    def update(lm):
        tile_max = jnp.max(lm, axis=1, keepdims=True)
        m_old = m_ref[...]
        m_new = jnp.maximum(m_old, tile_max)
        s_ref[...] = s_ref[...] * jnp.exp(m_old - m_new) + jnp.sum(
            jnp.exp(lm - m_new), axis=1, keepdims=True
        )
        m_ref[...] = m_new

    @pl.when(j < nvt - 1)
    def _mid():
        update(logits)

    @pl.when(j == nvt - 1)
    def _fin():
        iota = jax.lax.broadcasted_iota(jnp.int32, (S, tv), 1)
        update(jnp.where(j * tv + iota < V, logits, NEG_INF))
        row = jax.lax.broadcasted_iota(jnp.int32, (S, 1), 0)
        nll = (m_ref[...] + jnp.log(s_ref[...])) - ll_ref[...]
        nll = jnp.where(row < S - 1, nll, 0.0)
        loss_ref[0, 0] = jnp.sum(nll) * (1.0 / (S - 1))


def _decoder_loss(x, w, rows, tv=1024):
    nvt = (V + tv - 1) // tv
    out, loss = pl.pallas_call(
        functools.partial(_decoder_kernel, tv=tv, nvt=nvt),
        grid=(nvt,),
        in_specs=[
            pl.BlockSpec((S, D), lambda j: (0, 0)),
            pl.BlockSpec((tv, D), lambda j: (j, 0)),
            pl.BlockSpec((S, D), lambda j: (0, 0)),
        ],
        compiler_params=pltpu.CompilerParams(
            dimension_semantics=("arbitrary",),
        ),
        out_specs=[
            pl.BlockSpec((S, tv), lambda j: (0, j)),
            pl.BlockSpec(memory_space=pltpu.SMEM),
        ],
        out_shape=[
            jax.ShapeDtypeStruct((S, V), jnp.float32),
            jax.ShapeDtypeStruct((1, 1), jnp.float32),
        ],
        scratch_shapes=[
            pltpu.VMEM((S, 1), jnp.float32),
            pltpu.VMEM((S, 1), jnp.float32),
            pltpu.VMEM((S, 1), jnp.float32),
        ],
    )(x, w, rows)
    return out, loss[0, 0]


def kernel(inputs, chain_weights, noise_unit, responses, return_ops, params,
           decoder_w):
    labels = jnp.concatenate([inputs[0, 1:], jnp.zeros((1,), jnp.int32)])
    lab_rows = _sc_gather_rows(decoder_w, labels)

    w = _joining_weights(chain_weights, noise_unit, return_ops)
    x = _combine(w, responses.reshape(TOPK, S, D))

    for li, p in enumerate(params["layers"]):
        qkv = _qkv(
            x,
            p["in_proj_w"],
            p["in_proj_b"].reshape(1, 3 * D),
        )
        attn_out = _attention(qkv)
        last = li == len(params["layers"]) - 1
        x = _post(
            attn_out, x, p,
            out_dtype=jnp.bfloat16 if last else jnp.float32,
        )

    outputs, loss = _decoder_loss(x, decoder_w, lab_rows)
    return (outputs.reshape(1, S, V), loss)
